# baseline (speedup 1.0000x reference)
import jax
import jax.numpy as jnp
from jax.experimental import pallas as pl
from jax.experimental.pallas import tpu as pltpu

_B, _H, _W, _C = 16, 32, 32, 256
_CCHUNK = 128


def _body(row_ref, col_ref, pose_ref, pemb_ref, m_ref):
    q = pl.program_id(1)

    pv = pose_ref[0:1, :]
    prow = jnp.concatenate([pv, pv], axis=1)
    pemb_ref[...] = jnp.broadcast_to(prow, (_B, 2 * _C))

    @pl.when(q < 2)
    def _():
        c0 = q * _CCHUNK
        sl = col_ref[pl.ds(1, _W), pl.ds(c0, _CCHUNK)]
        t = jnp.transpose(sl, (1, 0))
        m_ref[...] = jnp.broadcast_to(t[None, :, None, :], (1, _CCHUNK, _H, _W))

    @pl.when(q >= 2)
    def _():
        c0 = (q - 2) * _CCHUNK
        sl = row_ref[pl.ds(1, _H), pl.ds(c0, _CCHUNK)]
        t = jnp.transpose(sl, (1, 0))
        m_ref[...] = jnp.broadcast_to(t[None, :, :, None], (1, _CCHUNK, _H, _W))


def kernel(x, row_embed, col_embed, pose_token_embed):
    del x
    grid = (_B, (2 * _C) // _CCHUNK)
    p_emb, m_emb = pl.pallas_call(
        _body,
        grid=grid,
        in_specs=[
            pl.BlockSpec(row_embed.shape, lambda b, q: (0, 0)),
            pl.BlockSpec(col_embed.shape, lambda b, q: (0, 0)),
            pl.BlockSpec(pose_token_embed.shape, lambda b, q: (0, 0)),
        ],
        out_specs=[
            pl.BlockSpec((_B, 2 * _C), lambda b, q: (0, 0)),
            pl.BlockSpec((1, _CCHUNK, _H, _W), lambda b, q: (b, q, 0, 0)),
        ],
        out_shape=[
            jax.ShapeDtypeStruct((_B, 2 * _C), jnp.float32),
            jax.ShapeDtypeStruct((_B, 2 * _C, _H, _W), jnp.float32),
        ],
    )(row_embed, col_embed, pose_token_embed)
    return (p_emb, m_emb)
